# Initial kernel scaffold; baseline (speedup 1.0000x reference)
import jax
import jax.numpy as jnp
from jax.experimental import pallas as pl


def kernel(right_input, disparity_samples):
    raise NotImplementedError("write your pallas kernel here")


# speedup vs baseline: 5.4045x; 5.4045x over previous
import functools

import jax
import jax.numpy as jnp
from jax import lax
from jax.experimental import pallas as pl
from jax.experimental.pallas import tpu as pltpu
from jax.experimental.pallas import tpu_sc as plsc

LANES = 16


@functools.lru_cache(maxsize=None)
def _make_warp(B, C, H, W):
    info = plsc.get_sparse_core_info()
    NW = info.num_cores * info.num_subcores
    NPAIRS = B * H
    assert NPAIRS % NW == 0
    PPW = NPAIRS // NW
    NCHUNK = W // LANES

    mesh = plsc.VectorSubcoreMesh(core_axis_name="c", subcore_axis_name="s")

    @functools.partial(
        pl.kernel,
        mesh=mesh,
        compiler_params=pltpu.CompilerParams(
            use_tc_tiling_on_sc=False, needs_layout_passes=False
        ),
        out_type=jax.ShapeDtypeStruct((B, C, H * W), jnp.float32),
        scratch_types=[
            pltpu.VMEM((W,), jnp.float32),
            pltpu.VMEM((C, W), jnp.float32),
            pltpu.VMEM((C, W), jnp.float32),
        ],
    )
    def warp(right_hbm, disp_hbm, out_hbm, disp_v, in_v, out_v):
        wid = lax.axis_index("s") * info.num_cores + lax.axis_index("c")

        def pair_body(k, carry):
            p = wid * PPW + k
            b = p // H
            h = p % H
            pltpu.sync_copy(disp_hbm.at[b, pl.ds(h * W, W)], disp_v)
            pltpu.sync_copy(right_hbm.at[b, :, pl.ds(h * W, W)], in_v)

            def chunk_body(j, carry2):
                wi = lax.iota(jnp.int32, 16) + j * LANES
                y = wi.astype(jnp.float32) + disp_v[pl.ds(j * LANES, LANES)]
                t = y.astype(jnp.int32).astype(jnp.float32)
                fl = jnp.where(t > y, t - 1.0, t)
                ia = jnp.clip(fl.astype(jnp.int32), 0, W - 1)
                ib = jnp.minimum(ia + 1, W - 1)
                inb = (y >= 0.0) & (y <= jnp.float32(W - 1))
                zero = jnp.zeros((LANES,), jnp.float32)
                wa = jnp.where(inb, (fl + 1.0) - y, zero)
                wb = jnp.where(inb, y - fl, zero)

                def c_body(c, carry3):
                    cs = jnp.full((LANES,), c, jnp.int32)
                    va = plsc.load_gather(in_v, [cs, ia])
                    vb = plsc.load_gather(in_v, [cs, ib])
                    plsc.store_scatter(out_v, [cs, wi], wa * va + wb * vb)
                    return carry3

                return lax.fori_loop(0, C, c_body, carry2)

            lax.fori_loop(0, NCHUNK, chunk_body, 0)
            pltpu.sync_copy(out_v, out_hbm.at[b, :, pl.ds(h * W, W)])
            return carry

        lax.fori_loop(0, PPW, pair_body, 0)

    return warp


def kernel(right_input, disparity_samples):
    B, C, H, W = right_input.shape
    warp = _make_warp(B, C, H, W)
    r2 = right_input.reshape(B, C, H * W)
    d2 = disparity_samples.reshape(B, H * W)
    out = warp(r2, d2)
    return out.reshape(B, C, H, W)
